# initial kernel scaffold (unmeasured)
import jax
import jax.numpy as jnp
from jax import lax
from jax.experimental import pallas as pl
from jax.experimental.pallas import tpu as pltpu

N_GLOBAL = 4096
EPS = 1e-5
CHUNK_M = 768
STAT_LANES = 128


def kernel(x, gamma, beta):
    m_per, n_per = x.shape
    num_chunks = m_per // CHUNK_M
    gamma2 = gamma.reshape(1, n_per)
    beta2 = beta.reshape(1, n_per)

    def body(x_ref, g_ref, b_ref, out_ref,
             stats_send, stats_recv, send_sem, recv_sem, credit_sem):
        i = pl.program_id(0)
        my_x = lax.axis_index("x")
        my_y = lax.axis_index("y")
        nbr = (my_x, 1 - my_y)

        @pl.when(i == 0)
        def _():
            barrier_sem = pltpu.get_barrier_semaphore()
            pl.semaphore_signal(
                barrier_sem, inc=1,
                device_id=nbr, device_id_type=pl.DeviceIdType.MESH,
            )
            pl.semaphore_wait(barrier_sem, 1)

        xv = x_ref[:, :]
        stats_send[:, 0:1] = jnp.sum(xv, axis=1, keepdims=True)
        stats_send[:, 1:2] = jnp.sum(xv * xv, axis=1, keepdims=True)

        @pl.when(i > 0)
        def _():
            pl.semaphore_wait(credit_sem, 1)

        rdma = pltpu.make_async_remote_copy(
            src_ref=stats_send,
            dst_ref=stats_recv,
            send_sem=send_sem,
            recv_sem=recv_sem,
            device_id=nbr,
            device_id_type=pl.DeviceIdType.MESH,
        )
        rdma.start()
        rdma.wait()

        tot_sum = stats_send[:, 0:1] + stats_recv[:, 0:1]
        tot_sq = stats_send[:, 1:2] + stats_recv[:, 1:2]
        mean = tot_sum / N_GLOBAL
        var = tot_sq / N_GLOBAL - mean * mean
        inv = lax.rsqrt(var + EPS)

        pl.semaphore_signal(
            credit_sem, inc=1,
            device_id=nbr, device_id_type=pl.DeviceIdType.MESH,
        )

        out_ref[:, :] = g_ref[:, :] * (xv - mean) * inv + b_ref[:, :]

    return pl.pallas_call(
        body,
        grid=(num_chunks,),
        out_shape=jax.ShapeDtypeStruct((m_per, n_per), x.dtype),
        in_specs=[
            pl.BlockSpec((CHUNK_M, n_per), lambda i: (i, 0)),
            pl.BlockSpec((1, n_per), lambda i: (0, 0)),
            pl.BlockSpec((1, n_per), lambda i: (0, 0)),
        ],
        out_specs=pl.BlockSpec((CHUNK_M, n_per), lambda i: (i, 0)),
        scratch_shapes=[
            pltpu.VMEM((CHUNK_M, STAT_LANES), jnp.float32),
            pltpu.VMEM((CHUNK_M, STAT_LANES), jnp.float32),
            pltpu.SemaphoreType.DMA,
            pltpu.SemaphoreType.DMA,
            pltpu.SemaphoreType.REGULAR,
        ],
        compiler_params=pltpu.CompilerParams(
            dimension_semantics=("arbitrary",),
            collective_id=0,
        ),
    )(x, gamma2, beta2)


# baseline (device time: 113649 ns/iter reference)
import jax
import jax.numpy as jnp
from jax import lax
from jax.experimental import pallas as pl
from jax.experimental.pallas import tpu as pltpu

N_GLOBAL = 4096
EPS = 1e-5
CHUNK_M = 512
STAT_LANES = 128


def kernel(x, gamma, beta):
    m_per, n_per = x.shape
    num_chunks = m_per // CHUNK_M
    gamma2 = gamma.reshape(1, n_per)
    beta2 = beta.reshape(1, n_per)

    def body(x_ref, g_ref, b_ref, out_ref,
             stats_send, stats_recv, send_sem, recv_sems):
        i = pl.program_id(0)
        my_x = lax.axis_index("x")
        my_y = lax.axis_index("y")
        nbr = (my_x, 1 - my_y)

        @pl.when(i == 0)
        def _():
            barrier_sem = pltpu.get_barrier_semaphore()
            pl.semaphore_signal(
                barrier_sem, inc=1,
                device_id=nbr, device_id_type=pl.DeviceIdType.MESH,
            )
            pl.semaphore_wait(barrier_sem, 1)

        xv = x_ref[:, :]
        stats_send[:, 0:1] = jnp.sum(xv, axis=1, keepdims=True)
        stats_send[:, 1:2] = jnp.sum(xv * xv, axis=1, keepdims=True)

        rdma = pltpu.make_async_remote_copy(
            src_ref=stats_send,
            dst_ref=stats_recv.at[i],
            send_sem=send_sem,
            recv_sem=recv_sems.at[i],
            device_id=nbr,
            device_id_type=pl.DeviceIdType.MESH,
        )
        rdma.start()
        rdma.wait()

        tot_sum = stats_send[:, 0:1] + stats_recv[i, :, 0:1]
        tot_sq = stats_send[:, 1:2] + stats_recv[i, :, 1:2]
        mean = tot_sum / N_GLOBAL
        var = tot_sq / N_GLOBAL - mean * mean
        inv = lax.rsqrt(var + EPS)

        out_ref[:, :] = g_ref[:, :] * (xv - mean) * inv + b_ref[:, :]

    return pl.pallas_call(
        body,
        grid=(num_chunks,),
        out_shape=jax.ShapeDtypeStruct((m_per, n_per), x.dtype),
        in_specs=[
            pl.BlockSpec((CHUNK_M, n_per), lambda i: (i, 0)),
            pl.BlockSpec((1, n_per), lambda i: (0, 0)),
            pl.BlockSpec((1, n_per), lambda i: (0, 0)),
        ],
        out_specs=pl.BlockSpec((CHUNK_M, n_per), lambda i: (i, 0)),
        scratch_shapes=[
            pltpu.VMEM((CHUNK_M, STAT_LANES), jnp.float32),
            pltpu.VMEM((num_chunks, CHUNK_M, STAT_LANES), jnp.float32),
            pltpu.SemaphoreType.DMA,
            pltpu.SemaphoreType.DMA((num_chunks,)),
        ],
        compiler_params=pltpu.CompilerParams(
            dimension_semantics=("arbitrary",),
            collective_id=0,
        ),
    )(x, gamma2, beta2)


# device time: 92365 ns/iter; 1.2304x vs baseline; 1.2304x over previous
import jax
import jax.numpy as jnp
from jax import lax
from jax.experimental import pallas as pl
from jax.experimental.pallas import tpu as pltpu

N_GLOBAL = 4096
EPS = 1e-5
CHUNK_M = 384
STAT_LANES = 128


def kernel(x, gamma, beta):
    m_per, n_per = x.shape
    nc = m_per // CHUNK_M
    gamma2 = gamma.reshape(1, n_per)
    beta2 = beta.reshape(1, n_per)

    def body(x_ref, g_ref, b_ref, out_ref,
             stats_mine, stats_recv, x_save,
             send_sem, recv_sems, copy_sem):
        i = pl.program_id(0)
        my_x = lax.axis_index("x")
        my_y = lax.axis_index("y")
        nbr = (my_x, 1 - my_y)

        @pl.when(i == 0)
        def _():
            barrier_sem = pltpu.get_barrier_semaphore()
            pl.semaphore_signal(
                barrier_sem, inc=1,
                device_id=nbr, device_id_type=pl.DeviceIdType.MESH,
            )
            pl.semaphore_wait(barrier_sem, 1)

        @pl.when(i < nc)
        def _():
            xv = x_ref[:, :]
            stats_mine[i, :, 0:1] = jnp.sum(xv, axis=1, keepdims=True)
            stats_mine[i, :, 1:2] = jnp.sum(xv * xv, axis=1, keepdims=True)
            rdma = pltpu.make_async_remote_copy(
                src_ref=stats_mine.at[i],
                dst_ref=stats_recv.at[i],
                send_sem=send_sem,
                recv_sem=recv_sems.at[i],
                device_id=nbr,
                device_id_type=pl.DeviceIdType.MESH,
            )
            rdma.start()
            cp = pltpu.make_async_copy(x_ref, x_save.at[lax.rem(i, 2)],
                                       copy_sem)
            cp.start()

        @pl.when(i > 0)
        def _():
            j = i - 1
            recv = pltpu.make_async_remote_copy(
                src_ref=stats_mine.at[j],
                dst_ref=stats_recv.at[j],
                send_sem=send_sem,
                recv_sem=recv_sems.at[j],
                device_id=nbr,
                device_id_type=pl.DeviceIdType.MESH,
            )
            recv.wait_recv()
            tot_sum = stats_mine[j, :, 0:1] + stats_recv[j, :, 0:1]
            tot_sq = stats_mine[j, :, 1:2] + stats_recv[j, :, 1:2]
            mean = tot_sum / N_GLOBAL
            var = tot_sq / N_GLOBAL - mean * mean
            inv = lax.rsqrt(var + EPS)
            xp = x_save[lax.rem(j, 2)]
            out_ref[:, :] = g_ref[:, :] * (xp - mean) * inv + b_ref[:, :]

        @pl.when(i < nc)
        def _():
            pltpu.make_async_copy(x_ref, x_save.at[lax.rem(i, 2)],
                                  copy_sem).wait()
            send_done = pltpu.make_async_remote_copy(
                src_ref=stats_mine.at[i],
                dst_ref=stats_recv.at[i],
                send_sem=send_sem,
                recv_sem=recv_sems.at[i],
                device_id=nbr,
                device_id_type=pl.DeviceIdType.MESH,
            )
            send_done.wait_send()

    return pl.pallas_call(
        body,
        grid=(nc + 1,),
        out_shape=jax.ShapeDtypeStruct((m_per, n_per), x.dtype),
        in_specs=[
            pl.BlockSpec((CHUNK_M, n_per), lambda i: (min_idx(i, nc - 1), 0)),
            pl.BlockSpec((1, n_per), lambda i: (0, 0)),
            pl.BlockSpec((1, n_per), lambda i: (0, 0)),
        ],
        out_specs=pl.BlockSpec((CHUNK_M, n_per), lambda i: (max_idx(i - 1), 0)),
        scratch_shapes=[
            pltpu.VMEM((nc, CHUNK_M, STAT_LANES), jnp.float32),
            pltpu.VMEM((nc, CHUNK_M, STAT_LANES), jnp.float32),
            pltpu.VMEM((2, CHUNK_M, n_per), jnp.float32),
            pltpu.SemaphoreType.DMA,
            pltpu.SemaphoreType.DMA((nc,)),
            pltpu.SemaphoreType.DMA,
        ],
        compiler_params=pltpu.CompilerParams(
            dimension_semantics=("arbitrary",),
            collective_id=0,
        ),
    )(x, gamma2, beta2)


def min_idx(i, cap):
    return jnp.minimum(i, cap)


def max_idx(i):
    return jnp.maximum(i, 0)


# device time: 90141 ns/iter; 1.2608x vs baseline; 1.0247x over previous
import jax
import jax.numpy as jnp
from jax import lax
from jax.experimental import pallas as pl
from jax.experimental.pallas import tpu as pltpu

N_GLOBAL = 4096
EPS = 1e-5
CHUNK_M = 512
STAT_LANES = 128


def kernel(x, gamma, beta):
    m_per, n_per = x.shape
    nc = m_per // CHUNK_M
    gamma2 = gamma.reshape(1, n_per)
    beta2 = beta.reshape(1, n_per)

    def fetch(x_ref, xbuf, xin_sems, k):
        return pltpu.make_async_copy(
            x_ref.at[pl.ds(k * CHUNK_M, CHUNK_M), :],
            xbuf.at[lax.rem(k, 3)],
            xin_sems.at[lax.rem(k, 3)],
        )

    def body(x_ref, g_ref, b_ref, out_ref,
             xbuf, stats_mine, stats_recv,
             xin_sems, send_sem, recv_sems):
        i = pl.program_id(0)
        my_x = lax.axis_index("x")
        my_y = lax.axis_index("y")
        nbr = (my_x, 1 - my_y)

        @pl.when(i == 0)
        def _():
            barrier_sem = pltpu.get_barrier_semaphore()
            pl.semaphore_signal(
                barrier_sem, inc=1,
                device_id=nbr, device_id_type=pl.DeviceIdType.MESH,
            )
            pl.semaphore_wait(barrier_sem, 1)
            fetch(x_ref, xbuf, xin_sems, 0).start()

        @pl.when(i < nc)
        def _():
            fetch(x_ref, xbuf, xin_sems, i).wait()
            xv = xbuf[lax.rem(i, 3)]
            stats_mine[i, :, 0:1] = jnp.sum(xv, axis=1, keepdims=True)
            stats_mine[i, :, 1:2] = jnp.sum(xv * xv, axis=1, keepdims=True)
            rdma = pltpu.make_async_remote_copy(
                src_ref=stats_mine.at[i],
                dst_ref=stats_recv.at[i],
                send_sem=send_sem,
                recv_sem=recv_sems.at[i],
                device_id=nbr,
                device_id_type=pl.DeviceIdType.MESH,
            )
            rdma.start()

            @pl.when(i + 1 < nc)
            def _():
                fetch(x_ref, xbuf, xin_sems, i + 1).start()

        @pl.when(i > 0)
        def _():
            j = i - 1
            recv = pltpu.make_async_remote_copy(
                src_ref=stats_mine.at[j],
                dst_ref=stats_recv.at[j],
                send_sem=send_sem,
                recv_sem=recv_sems.at[j],
                device_id=nbr,
                device_id_type=pl.DeviceIdType.MESH,
            )
            recv.wait_recv()
            tot_sum = stats_mine[j, :, 0:1] + stats_recv[j, :, 0:1]
            tot_sq = stats_mine[j, :, 1:2] + stats_recv[j, :, 1:2]
            mean = tot_sum / N_GLOBAL
            var = tot_sq / N_GLOBAL - mean * mean
            inv = lax.rsqrt(var + EPS)
            xp = xbuf[lax.rem(j, 3)]
            out_ref[:, :] = g_ref[:, :] * (xp - mean) * inv + b_ref[:, :]

        @pl.when(i < nc)
        def _():
            pltpu.make_async_remote_copy(
                src_ref=stats_mine.at[i],
                dst_ref=stats_recv.at[i],
                send_sem=send_sem,
                recv_sem=recv_sems.at[i],
                device_id=nbr,
                device_id_type=pl.DeviceIdType.MESH,
            ).wait_send()

    return pl.pallas_call(
        body,
        grid=(nc + 1,),
        out_shape=jax.ShapeDtypeStruct((m_per, n_per), x.dtype),
        in_specs=[
            pl.BlockSpec(memory_space=pl.ANY),
            pl.BlockSpec((1, n_per), lambda i: (0, 0)),
            pl.BlockSpec((1, n_per), lambda i: (0, 0)),
        ],
        out_specs=pl.BlockSpec(
            (CHUNK_M, n_per), lambda i: (jnp.maximum(i - 1, 0), 0)
        ),
        scratch_shapes=[
            pltpu.VMEM((3, CHUNK_M, n_per), jnp.float32),
            pltpu.VMEM((nc, CHUNK_M, STAT_LANES), jnp.float32),
            pltpu.VMEM((nc, CHUNK_M, STAT_LANES), jnp.float32),
            pltpu.SemaphoreType.DMA((3,)),
            pltpu.SemaphoreType.DMA,
            pltpu.SemaphoreType.DMA((nc,)),
        ],
        compiler_params=pltpu.CompilerParams(
            dimension_semantics=("arbitrary",),
            collective_id=0,
        ),
    )(x, gamma2, beta2)


# device time: 52527 ns/iter; 2.1636x vs baseline; 1.7161x over previous
import jax
import jax.numpy as jnp
from jax import lax
from jax.experimental import pallas as pl
from jax.experimental.pallas import tpu as pltpu

N_GLOBAL = 4096
EPS = 1e-5
STAT_ROWS = 8


def _fetcher(x_ref, xbuf, xin_sems, chunk_m, k):
    return pltpu.make_async_copy(
        x_ref.at[pl.ds(k * chunk_m, chunk_m), :],
        xbuf.at[lax.rem(k, 6)],
        xin_sems.at[lax.rem(k, 6)],
    )


def _stats_kernel(x):
    m_per, n_per = x.shape
    chunk_m = 512
    nc = m_per // chunk_m

    def body(x_ref, st_ref, xbuf, stats_mine, stats_recv,
             xin_sems, send_sems, recv_sems):
        i = pl.program_id(0)
        my_x = lax.axis_index("x")
        my_y = lax.axis_index("y")
        nbr = (my_x, 1 - my_y)

        def fetch(k):
            return _fetcher(x_ref, xbuf, xin_sems, chunk_m, k)

        @pl.when(i == 0)
        def _():
            barrier_sem = pltpu.get_barrier_semaphore()
            pl.semaphore_signal(
                barrier_sem, inc=1,
                device_id=nbr, device_id_type=pl.DeviceIdType.MESH,
            )
            pl.semaphore_wait(barrier_sem, 1)
            fetch(0).start()
            fetch(1).start()
            fetch(2).start()

        @pl.when(i < nc)
        def _():
            fetch(i).wait()
            xv = xbuf[lax.rem(i, 6)]
            s = jnp.sum(xv, axis=1)
            q = jnp.sum(xv * xv, axis=1)
            sm = stats_mine.at[lax.rem(i, 4)]
            sm[0:1, :] = s.reshape(1, chunk_m)
            sm[1:2, :] = q.reshape(1, chunk_m)
            rdma = pltpu.make_async_remote_copy(
                src_ref=stats_mine.at[lax.rem(i, 4)],
                dst_ref=stats_recv.at[i],
                send_sem=send_sems.at[lax.rem(i, 4)],
                recv_sem=recv_sems.at[i],
                device_id=nbr,
                device_id_type=pl.DeviceIdType.MESH,
            )
            rdma.start()

            @pl.when(i + 3 < nc)
            def _():
                fetch(i + 3).start()

        @pl.when(i > 1)
        def _():
            j = i - 2
            recv = pltpu.make_async_remote_copy(
                src_ref=stats_mine.at[lax.rem(j, 4)],
                dst_ref=stats_recv.at[j],
                send_sem=send_sems.at[lax.rem(j, 4)],
                recv_sem=recv_sems.at[j],
                device_id=nbr,
                device_id_type=pl.DeviceIdType.MESH,
            )
            recv.wait_send()
            recv.wait_recv()
            tot = stats_mine[lax.rem(j, 4)] + stats_recv[j]
            mean_r = tot[0:1, :] / N_GLOBAL
            var_r = tot[1:2, :] / N_GLOBAL - mean_r * mean_r
            inv_r = lax.rsqrt(var_r + EPS)
            st_ref[0:1, pl.ds(j * chunk_m, chunk_m)] = mean_r
            st_ref[1:2, pl.ds(j * chunk_m, chunk_m)] = inv_r

    return pl.pallas_call(
        body,
        grid=(nc + 2,),
        out_shape=jax.ShapeDtypeStruct((STAT_ROWS, m_per), jnp.float32),
        in_specs=[pl.BlockSpec(memory_space=pl.ANY)],
        out_specs=pl.BlockSpec((STAT_ROWS, m_per), lambda i: (0, 0)),
        scratch_shapes=[
            pltpu.VMEM((6, chunk_m, n_per), jnp.float32),
            pltpu.VMEM((4, STAT_ROWS, chunk_m), jnp.float32),
            pltpu.VMEM((nc, STAT_ROWS, chunk_m), jnp.float32),
            pltpu.SemaphoreType.DMA((6,)),
            pltpu.SemaphoreType.DMA((4,)),
            pltpu.SemaphoreType.DMA((nc,)),
        ],
        compiler_params=pltpu.CompilerParams(
            dimension_semantics=("arbitrary",),
            collective_id=0,
        ),
    )(x)


def _normalize_kernel(x, stats, gamma2, beta2):
    m_per, n_per = x.shape
    chunk_m = 384
    nc = m_per // chunk_m

    def body(x_ref, st_ref, g_ref, b_ref, out_ref,
             xbuf, obuf, xin_sems, oout_sems):
        j = pl.program_id(0)

        def fetch(k):
            return _fetcher(x_ref, xbuf, xin_sems, chunk_m, k)

        def wb(k):
            return pltpu.make_async_copy(
                obuf.at[lax.rem(k, 3)],
                out_ref.at[pl.ds(k * chunk_m, chunk_m), :],
                oout_sems.at[lax.rem(k, 3)],
            )

        @pl.when(j == 0)
        def _():
            fetch(0).start()
            fetch(1).start()
            fetch(2).start()

        @pl.when(j + 3 < nc)
        def _():
            fetch(j + 3).start()

        fetch(j).wait()
        xv = xbuf[lax.rem(j, 6)]
        mean_r = st_ref[0:1, pl.ds(j * chunk_m, chunk_m)]
        inv_r = st_ref[1:2, pl.ds(j * chunk_m, chunk_m)]
        mean_c = mean_r.reshape(chunk_m, 1)
        inv_c = inv_r.reshape(chunk_m, 1)

        @pl.when(j > 2)
        def _():
            wb(j - 3).wait()

        obuf[lax.rem(j, 3)] = (
            g_ref[:, :] * (xv - mean_c) * inv_c + b_ref[:, :]
        )
        wb(j).start()

        @pl.when(j == nc - 1)
        def _():
            wb(nc - 3).wait()
            wb(nc - 2).wait()
            wb(nc - 1).wait()

    return pl.pallas_call(
        body,
        grid=(nc,),
        out_shape=jax.ShapeDtypeStruct((m_per, n_per), x.dtype),
        in_specs=[
            pl.BlockSpec(memory_space=pl.ANY),
            pl.BlockSpec((STAT_ROWS, m_per), lambda j: (0, 0)),
            pl.BlockSpec((1, n_per), lambda j: (0, 0)),
            pl.BlockSpec((1, n_per), lambda j: (0, 0)),
        ],
        out_specs=pl.BlockSpec(memory_space=pl.ANY),
        scratch_shapes=[
            pltpu.VMEM((6, chunk_m, n_per), jnp.float32),
            pltpu.VMEM((3, chunk_m, n_per), jnp.float32),
            pltpu.SemaphoreType.DMA((6,)),
            pltpu.SemaphoreType.DMA((3,)),
        ],
        compiler_params=pltpu.CompilerParams(
            dimension_semantics=("arbitrary",),
        ),
    )(x, stats, gamma2, beta2)


def kernel(x, gamma, beta):
    m_per, n_per = x.shape
    gamma2 = gamma.reshape(1, n_per)
    beta2 = beta.reshape(1, n_per)
    stats = _stats_kernel(x)
    return _normalize_kernel(x, stats, gamma2, beta2)
